# Initial kernel scaffold; baseline (speedup 1.0000x reference)
import jax
import jax.numpy as jnp
from jax.experimental import pallas as pl


def kernel(X, ref_a, ref_b, W, W_0, b):
    raise NotImplementedError("write your pallas kernel here")


# speedup vs baseline: 4.3576x; 4.3576x over previous
import functools

import jax
import jax.numpy as jnp
from jax import lax
from jax.experimental import pallas as pl
from jax.experimental.pallas import tpu as pltpu
from jax.experimental.pallas import tpu_sc as plsc


def _matmul_body(x_ref, w_ref, y_ref):
    y_ref[...] = jnp.dot(x_ref[...], w_ref[0], preferred_element_type=jnp.float32)


def _relation_matmuls(X, W, bm):
    N, D = X.shape
    R, _, U = W.shape
    nm = N // bm
    return pl.pallas_call(
        _matmul_body,
        grid=(R, nm),
        in_specs=[
            pl.BlockSpec((bm, D), lambda r, m: (m, 0)),
            pl.BlockSpec((1, D, U), lambda r, m: (r, 0, 0)),
        ],
        out_specs=pl.BlockSpec((bm, U), lambda r, m: (r * nm + m, 0)),
        out_shape=jax.ShapeDtypeStruct((R * N, U), jnp.float32),
    )(X, W)


def _combine_body(p_ref, x_ref, w0_ref, b_ref, o_ref):
    z = jnp.dot(x_ref[...], w0_ref[...], preferred_element_type=jnp.float32)
    acc = p_ref[0] + p_ref[1] + z + b_ref[...]
    o_ref[...] = jnp.maximum(acc, 0.0)


def _combine(P, X, W_0, b, bm):
    N, D = X.shape
    U = W_0.shape[1]
    nm = N // bm
    return pl.pallas_call(
        _combine_body,
        grid=(nm,),
        in_specs=[
            pl.BlockSpec((2, bm, U), lambda m: (0, m, 0)),
            pl.BlockSpec((bm, D), lambda m: (m, 0)),
            pl.BlockSpec((D, U), lambda m: (0, 0)),
            pl.BlockSpec((1, U), lambda m: (0, 0)),
        ],
        out_specs=pl.BlockSpec((bm, U), lambda m: (m, 0)),
        out_shape=jax.ShapeDtypeStruct((N, U), jnp.float32),
    )(P, X, W_0, b.reshape(1, U))


def _make_scatter(N_acc, U, n_sc, n_sub, chunk, cpw, rpt):
    mesh = plsc.VectorSubcoreMesh(core_axis_name="c", subcore_axis_name="s")

    @functools.partial(
        pl.kernel,
        out_type=jax.ShapeDtypeStruct((n_sc, N_acc, U), jnp.float32),
        mesh=mesh,
        scratch_types=[
            pltpu.VMEM((chunk,), jnp.int32),
            pltpu.VMEM((chunk,), jnp.int32),
            pltpu.VMEM((chunk, U), jnp.float32),
            pltpu.VMEM_SHARED((N_acc, U), jnp.float32),
            pltpu.SemaphoreType.DMA,
        ],
    )
    def scatter_kernel(y_hbm, src_hbm, dst_hbm, zeros_hbm, out_hbm,
                       sidx, didx, rows, acc, sem):
        c = lax.axis_index("c")
        s = lax.axis_index("s")
        wid = s * n_sc + c
        pltpu.sync_copy(zeros_hbm, acc.at[pl.ds(s * rpt, rpt)])
        plsc.subcore_barrier()

        def step(j, carry):
            base = (wid * cpw + j) * chunk
            pltpu.sync_copy(src_hbm.at[pl.ds(base, chunk)], sidx)
            pltpu.sync_copy(dst_hbm.at[pl.ds(base, chunk)], didx)
            pltpu.async_copy(y_hbm.at[sidx], rows, sem).wait()
            pltpu.sync_copy(rows, acc.at[didx], add=True)
            return carry

        lax.fori_loop(0, cpw, step, 0)
        plsc.subcore_barrier()
        pltpu.sync_copy(acc.at[pl.ds(s * rpt, rpt)],
                        out_hbm.at[c, pl.ds(s * rpt, rpt)])

    return scatter_kernel


def kernel(X, ref_a, ref_b, W, W_0, b):
    N, D = X.shape
    R, _, U = W.shape
    E = ref_a.shape[1]

    info = plsc.get_sparse_core_info()
    n_sc, n_sub = info.num_cores, info.num_subcores
    nw = n_sc * n_sub
    chunk = 128
    rpt = -(-N // n_sub)
    rpt = ((rpt + 7) // 8) * 8
    N_acc = n_sub * rpt
    if N_acc == N:
        rpt += 8
        N_acc = n_sub * rpt
    dst_pad = N

    offs = (jnp.arange(R, dtype=jnp.int32) * N)[:, None]
    srcs = jnp.concatenate([(ref_b + offs).reshape(-1), (ref_a + offs).reshape(-1)])
    dsts = jnp.concatenate([ref_a.reshape(-1), ref_b.reshape(-1)])
    total = 2 * R * E
    cpw = -(-total // (nw * chunk))
    pad = nw * cpw * chunk - total
    if pad:
        srcs = jnp.concatenate([srcs, jnp.zeros((pad,), jnp.int32)])
        dsts = jnp.concatenate([dsts, jnp.full((pad,), dst_pad, jnp.int32)])

    bm = 400
    Y = _relation_matmuls(X, W, bm)

    zeros_hbm = jnp.zeros((rpt, U), jnp.float32)
    P = _make_scatter(N_acc, U, n_sc, n_sub, chunk, cpw, rpt)(
        Y, srcs, dsts, zeros_hbm)

    return _combine(P, X, W_0, b, bm)
